# Initial kernel scaffold; baseline (speedup 1.0000x reference)
import jax
import jax.numpy as jnp
from jax.experimental import pallas as pl


def kernel(x, offsets):
    raise NotImplementedError("write your pallas kernel here")


# speedup vs baseline: 1.4594x; 1.4594x over previous
import jax
import jax.numpy as jnp
from jax.experimental import pallas as pl
from jax.experimental.pallas import tpu as pltpu

_EPS = 1.0
_C = 32
_I = 32
_KH = 3
_KW = 3
_K = _C * _KH * _KW
_HP = 130
_WP = 130
_M = _HP * _WP
_PAD_B = 136
_PAD_A = 164
_MT = _PAD_B + _M + _PAD_A
_CHUNK = 4232
_NCH = 4
_MO = _NCH * _CHUNK
_HALO = _WP + 1
_XS_LEN = _CHUNK + 2 * _HALO
_OFFS = tuple(kh * _WP + kw for kh in range(_KH) for kw in range(_KW))


def _mex_kernel(x_ref, off_ref, o_ref):
    c = pl.program_id(1)
    start = c * _CHUNK + (_PAD_B - _HALO)
    xs = x_ref[0, pl.ds(start, _XS_LEN), :]
    gmax = jnp.max(xs)
    e = jnp.exp(xs - gmax)
    off = off_ref[...]
    mo = jnp.max(off, axis=0, keepdims=True)
    wt = jnp.exp(off - mo)
    p = jnp.concatenate([e[o:o + _CHUNK, :] for o in _OFFS], axis=1)
    u = jnp.dot(p, wt, preferred_element_type=jnp.float32)
    o_ref[0] = gmax + mo + (jnp.log(u) - jnp.log(jnp.float32(_K))) / _EPS


def kernel(x, offsets):
    n, ch, h, w = x.shape
    xt = jnp.transpose(x, (0, 2, 3, 1))
    xp = jnp.pad(xt, ((0, 0), (1, 1), (1, 1), (0, 0)))
    xf = xp.reshape(n, _M, ch)
    xf = jnp.pad(xf, ((0, 0), (_PAD_B, _PAD_A), (0, 0)))
    offt = (offsets.reshape(_I, _C, _KH, _KW)
            .transpose(2, 3, 1, 0).reshape(_K, _I))
    of = pl.pallas_call(
        _mex_kernel,
        out_shape=jax.ShapeDtypeStruct((n, _MO, _I), jnp.float32),
        grid=(n, _NCH),
        in_specs=[
            pl.BlockSpec((1, _MT, ch), lambda i, j: (i, 0, 0)),
            pl.BlockSpec((_K, _I), lambda i, j: (0, 0)),
        ],
        out_specs=pl.BlockSpec((1, _CHUNK, _I), lambda i, j: (i, j, 0)),
        compiler_params=pltpu.CompilerParams(
            dimension_semantics=("parallel", "arbitrary"),
            vmem_limit_bytes=50 * 1024 * 1024,
        ),
        name="mex_pool",
    )(xf, offt)
    of = of[:, :_M, :].reshape(n, _HP, _WP, _I)[:, 1:h + 1, 1:w + 1, :]
    return of.transpose(0, 3, 1, 2)


# speedup vs baseline: 5.7519x; 3.9411x over previous
import jax
import jax.numpy as jnp
from jax import lax
from jax.experimental import pallas as pl
from jax.experimental.pallas import tpu as pltpu

_EPS = 1.0
_C = 32
_I = 32
_KH = 3
_KW = 3
_K = _C * _KH * _KW
_W = 128
_CHUNK = 4096
_G = 256
_XS = _CHUNK + 2 * _G
_OFFS = tuple((kh - 1) * _W + (kw - 1) for kh in range(_KH) for kw in range(_KW))


def _mex_kernel(x_ref, off_ref, o_ref, xs_ref):
    c = pl.program_id(1)
    nch = pl.num_programs(1)
    xs_ref[:, :_G] = jnp.zeros((_C, _G), jnp.float32)
    xs_ref[:, _XS - _G:] = jnp.zeros((_C, _G), jnp.float32)
    xs_ref[:, _G:_G + _CHUNK] = x_ref[0, :, pl.ds(c * _CHUNK, _CHUNK)]

    @pl.when(c > 0)
    def _():
        xs_ref[:, :_G] = x_ref[0, :, pl.ds(c * _CHUNK - _G, _G)]

    @pl.when(c < nch - 1)
    def _():
        xs_ref[:, _XS - _G:] = x_ref[0, :, pl.ds((c + 1) * _CHUNK, _G)]

    xs = xs_ref[...]
    gmax = jnp.max(xs)
    e = jnp.exp(xs - gmax)
    pv = jnp.exp(-gmax)

    col = lax.broadcasted_iota(jnp.int32, (_C, _CHUNK), 1) % _W
    mask_l = col == 0
    mask_r = col == _W - 1

    taps = []
    for t, off in enumerate(_OFFS):
        sl = e[:, _G + off:_G + off + _CHUNK]
        kw = t % _KW
        if kw == 0:
            sl = jnp.where(mask_l, pv, sl)
        elif kw == 2:
            sl = jnp.where(mask_r, pv, sl)
        taps.append(sl)
    p = jnp.concatenate(taps, axis=0)

    off = off_ref[...]
    mo = jnp.max(off, axis=1, keepdims=True)
    wt = jnp.exp(off - mo)
    u = jnp.dot(wt, p, preferred_element_type=jnp.float32)
    o_ref[0] = gmax + mo + (jnp.log(u) - jnp.log(jnp.float32(_K))) / _EPS


def kernel(x, offsets):
    n, ch, h, w = x.shape
    m = h * w
    xf = x.reshape(n, ch, m)
    offt = (offsets.reshape(_I, _C, _KH * _KW)
            .transpose(0, 2, 1).reshape(_I, _K))
    of = pl.pallas_call(
        _mex_kernel,
        out_shape=jax.ShapeDtypeStruct((n, _I, m), jnp.float32),
        grid=(n, m // _CHUNK),
        in_specs=[
            pl.BlockSpec((1, ch, m), lambda i, j: (i, 0, 0)),
            pl.BlockSpec((_I, _K), lambda i, j: (0, 0)),
        ],
        out_specs=pl.BlockSpec((1, _I, _CHUNK), lambda i, j: (i, 0, j)),
        scratch_shapes=[pltpu.VMEM((_C, _XS), jnp.float32)],
        compiler_params=pltpu.CompilerParams(
            dimension_semantics=("parallel", "arbitrary"),
        ),
        name="mex_pool",
    )(xf, offt)
    return of.reshape(n, _I, h, w)


# speedup vs baseline: 11.3212x; 1.9683x over previous
import jax
import jax.numpy as jnp
from jax import lax
from jax.experimental import pallas as pl
from jax.experimental.pallas import tpu as pltpu

_EPS = 1.0
_C = 32
_I = 32
_KH = 3
_KW = 3
_K = _C * _KH * _KW
_H = 128
_W = 128
_M = _H * _W
_CHUNK = 4096
_NCH = _M // _CHUNK
_HB = _CHUNK // _W
_G = 256
_OFFS = tuple((kh - 1) * _W + (kw - 1) for kh in range(_KH) for kw in range(_KW))


def _mex_kernel(x_ref, off_ref, o_ref, xs_ref):
    c = pl.program_id(1)

    @pl.when(c == 0)
    def _():
        xs_ref[:, :_G] = jnp.zeros((_C, _G), jnp.float32)
        xs_ref[:, _G + _M:] = jnp.zeros((_C, _G), jnp.float32)
        xs_ref[:, _G:_G + _M] = x_ref[0].reshape(_C, _M)

    xsv = xs_ref[:, pl.ds(c * _CHUNK, _CHUNK + 2 * _G)]
    gmax = jnp.max(xsv)
    e = jnp.exp(xsv - gmax)
    pv = jnp.exp(-gmax)

    col = lax.broadcasted_iota(jnp.int32, (_C, _CHUNK), 1) % _W
    mask_l = col == 0
    mask_r = col == _W - 1

    taps = []
    for t, off in enumerate(_OFFS):
        sl = e[:, _G + off:_G + off + _CHUNK]
        kw = t % _KW
        if kw == 0:
            sl = jnp.where(mask_l, pv, sl)
        elif kw == 2:
            sl = jnp.where(mask_r, pv, sl)
        taps.append(sl)
    p = jnp.concatenate(taps, axis=0)

    off = off_ref[...]
    mo = jnp.max(off, axis=1, keepdims=True)
    wt = jnp.exp(off - mo)
    u = jnp.dot(wt, p, preferred_element_type=jnp.float32)
    res = gmax + mo + (jnp.log(u) - jnp.log(jnp.float32(_K))) / _EPS
    o_ref[0] = res.reshape(_I, _HB, _W)


def kernel(x, offsets):
    n, ch, h, w = x.shape
    offt = (offsets.reshape(_I, _C, _KH * _KW)
            .transpose(0, 2, 1).reshape(_I, _K))
    return pl.pallas_call(
        _mex_kernel,
        out_shape=jax.ShapeDtypeStruct((n, _I, h, w), jnp.float32),
        grid=(n, _NCH),
        in_specs=[
            pl.BlockSpec((1, ch, h, w), lambda i, j: (i, 0, 0, 0)),
            pl.BlockSpec((_I, _K), lambda i, j: (0, 0)),
        ],
        out_specs=pl.BlockSpec((1, _I, _HB, _W), lambda i, j: (i, 0, j, 0)),
        scratch_shapes=[pltpu.VMEM((_C, _M + 2 * _G), jnp.float32)],
        compiler_params=pltpu.CompilerParams(
            dimension_semantics=("parallel", "arbitrary"),
        ),
        name="mex_pool",
    )(x, offt)


# speedup vs baseline: 11.3721x; 1.0045x over previous
import jax
import jax.numpy as jnp
from jax import lax
from jax.experimental import pallas as pl
from jax.experimental.pallas import tpu as pltpu

_EPS = 1.0
_C = 32
_I = 32
_KH = 3
_KW = 3
_K = _C * _KH * _KW
_H = 128
_W = 128
_M = _H * _W
_CHUNK = 4096
_NCH = _M // _CHUNK
_HB = _CHUNK // _W
_G = 256
_OFFS = tuple((kh - 1) * _W + (kw - 1) for kh in range(_KH) for kw in range(_KW))


def _mex_kernel(x_ref, off_ref, o_ref, xs_ref):
    c = pl.program_id(1)

    @pl.when(c == 0)
    def _():
        xs_ref[:, :_G] = jnp.zeros((_C, _G), jnp.float32)
        xs_ref[:, _G + _M:] = jnp.zeros((_C, _G), jnp.float32)

    base = _G + c * _CHUNK
    xs_ref[:, pl.ds(base, _CHUNK)] = (
        x_ref[0, :, pl.ds(c * _HB, _HB), :].reshape(_C, _CHUNK))

    @pl.when(c < _NCH - 1)
    def _():
        xs_ref[:, pl.ds(base + _CHUNK, 2 * _W)] = (
            x_ref[0, :, pl.ds((c + 1) * _HB, 2), :].reshape(_C, 2 * _W))

    xsv = xs_ref[:, pl.ds(c * _CHUNK, _CHUNK + 2 * _G)]
    gmax = jnp.max(xsv)
    e = jnp.exp(xsv - gmax)
    pv = jnp.exp(-gmax)

    col = lax.broadcasted_iota(jnp.int32, (_C, _CHUNK), 1) % _W
    mask_l = col == 0
    mask_r = col == _W - 1

    taps = []
    for t, off in enumerate(_OFFS):
        sl = e[:, _G + off:_G + off + _CHUNK]
        kw = t % _KW
        if kw == 0:
            sl = jnp.where(mask_l, pv, sl)
        elif kw == 2:
            sl = jnp.where(mask_r, pv, sl)
        taps.append(sl)
    p = jnp.concatenate(taps, axis=0)

    off = off_ref[...]
    mo = jnp.max(off, axis=1, keepdims=True)
    wt = jnp.exp(off - mo)
    u = jnp.dot(wt, p, preferred_element_type=jnp.float32)
    res = gmax + mo + (jnp.log(u) - jnp.log(jnp.float32(_K))) / _EPS
    o_ref[0] = res.reshape(_I, _HB, _W)


def kernel(x, offsets):
    n, ch, h, w = x.shape
    offt = (offsets.reshape(_I, _C, _KH * _KW)
            .transpose(0, 2, 1).reshape(_I, _K))
    return pl.pallas_call(
        _mex_kernel,
        out_shape=jax.ShapeDtypeStruct((n, _I, h, w), jnp.float32),
        grid=(n, _NCH),
        in_specs=[
            pl.BlockSpec((1, ch, h, w), lambda i, j: (i, 0, 0, 0)),
            pl.BlockSpec((_I, _K), lambda i, j: (0, 0)),
        ],
        out_specs=pl.BlockSpec((1, _I, _HB, _W), lambda i, j: (i, 0, j, 0)),
        scratch_shapes=[pltpu.VMEM((_C, _M + 2 * _G), jnp.float32)],
        compiler_params=pltpu.CompilerParams(
            dimension_semantics=("parallel", "arbitrary"),
        ),
        name="mex_pool",
    )(x, offt)


# speedup vs baseline: 14.3850x; 1.2649x over previous
import jax
import jax.numpy as jnp
from jax import lax
from jax.experimental import pallas as pl
from jax.experimental.pallas import tpu as pltpu

_EPS = 1.0
_C = 32
_I = 32
_KH = 3
_KW = 3
_K = _C * _KH * _KW
_H = 128
_W = 128
_M = _H * _W
_CHUNK = 8192
_NCH = _M // _CHUNK
_HB = _CHUNK // _W
_G = 256
_OFFS = tuple((kh - 1) * _W + (kw - 1) for kh in range(_KH) for kw in range(_KW))


def _mex_kernel(x_ref, off_ref, o_ref, xs_ref):
    c = pl.program_id(1)

    @pl.when(c == 0)
    def _():
        xs_ref[:, :_G] = jnp.zeros((_C, _G), jnp.float32)
        xs_ref[:, _G + _M:] = jnp.zeros((_C, _G), jnp.float32)

    base = _G + c * _CHUNK
    xs_ref[:, pl.ds(base, _CHUNK)] = (
        x_ref[0, :, pl.ds(c * _HB, _HB), :].reshape(_C, _CHUNK))

    @pl.when(c < _NCH - 1)
    def _():
        xs_ref[:, pl.ds(base + _CHUNK, 2 * _W)] = (
            x_ref[0, :, pl.ds((c + 1) * _HB, 2), :].reshape(_C, 2 * _W))

    xsv = xs_ref[:, pl.ds(c * _CHUNK, _CHUNK + 2 * _G)]
    gmax = jnp.max(xsv)
    e = jnp.exp(xsv - gmax)
    pv = jnp.exp(-gmax)

    col = lax.broadcasted_iota(jnp.int32, (_C, _CHUNK), 1) % _W
    mask_l = col == 0
    mask_r = col == _W - 1

    taps = []
    for t, off in enumerate(_OFFS):
        sl = e[:, _G + off:_G + off + _CHUNK]
        kw = t % _KW
        if kw == 0:
            sl = jnp.where(mask_l, pv, sl)
        elif kw == 2:
            sl = jnp.where(mask_r, pv, sl)
        taps.append(sl)
    p = jnp.concatenate(taps, axis=0)

    off = off_ref[...]
    mo = jnp.max(off, axis=1, keepdims=True)
    wt = jnp.exp(off - mo)
    u = jnp.dot(wt, p, preferred_element_type=jnp.float32)
    res = gmax + mo + (jnp.log(u) - jnp.log(jnp.float32(_K))) / _EPS
    o_ref[0] = res.reshape(_I, _HB, _W)


def kernel(x, offsets):
    n, ch, h, w = x.shape
    offt = (offsets.reshape(_I, _C, _KH * _KW)
            .transpose(0, 2, 1).reshape(_I, _K))
    return pl.pallas_call(
        _mex_kernel,
        out_shape=jax.ShapeDtypeStruct((n, _I, h, w), jnp.float32),
        grid=(n, _NCH),
        in_specs=[
            pl.BlockSpec((1, ch, h, w), lambda i, j: (i, 0, 0, 0)),
            pl.BlockSpec((_I, _K), lambda i, j: (0, 0)),
        ],
        out_specs=pl.BlockSpec((1, _I, _HB, _W), lambda i, j: (i, 0, j, 0)),
        scratch_shapes=[pltpu.VMEM((_C, _M + 2 * _G), jnp.float32)],
        compiler_params=pltpu.CompilerParams(
            dimension_semantics=("parallel", "arbitrary"),
        ),
        name="mex_pool",
    )(x, offt)


# speedup vs baseline: 17.3663x; 1.2073x over previous
import jax
import jax.numpy as jnp
from jax import lax
from jax.experimental import pallas as pl
from jax.experimental.pallas import tpu as pltpu

_EPS = 1.0
_C = 32
_I = 32
_KH = 3
_KW = 3
_K = _C * _KH * _KW
_H = 128
_W = 128
_M = _H * _W
_CHUNK = 16384
_NCH = _M // _CHUNK
_HB = _CHUNK // _W
_G = 256
_OFFS = tuple((kh - 1) * _W + (kw - 1) for kh in range(_KH) for kw in range(_KW))


def _mex_kernel(x_ref, off_ref, o_ref, xs_ref):
    c = pl.program_id(1)

    @pl.when(c == 0)
    def _():
        xs_ref[:, :_G] = jnp.zeros((_C, _G), jnp.float32)
        xs_ref[:, _G + _M:] = jnp.zeros((_C, _G), jnp.float32)

    base = _G + c * _CHUNK
    xs_ref[:, pl.ds(base, _CHUNK)] = (
        x_ref[0, :, pl.ds(c * _HB, _HB), :].reshape(_C, _CHUNK))

    @pl.when(c < _NCH - 1)
    def _():
        xs_ref[:, pl.ds(base + _CHUNK, 2 * _W)] = (
            x_ref[0, :, pl.ds((c + 1) * _HB, 2), :].reshape(_C, 2 * _W))

    xsv = xs_ref[:, pl.ds(c * _CHUNK, _CHUNK + 2 * _G)]
    gmax = jnp.max(xsv)
    e = jnp.exp(xsv - gmax)
    pv = jnp.exp(-gmax)

    col = lax.broadcasted_iota(jnp.int32, (_C, _CHUNK), 1) % _W
    mask_l = col == 0
    mask_r = col == _W - 1

    taps = []
    for t, off in enumerate(_OFFS):
        sl = e[:, _G + off:_G + off + _CHUNK]
        kw = t % _KW
        if kw == 0:
            sl = jnp.where(mask_l, pv, sl)
        elif kw == 2:
            sl = jnp.where(mask_r, pv, sl)
        taps.append(sl)
    p = jnp.concatenate(taps, axis=0)

    off = off_ref[...]
    mo = jnp.max(off, axis=1, keepdims=True)
    wt = jnp.exp(off - mo)
    u = jnp.dot(wt, p, preferred_element_type=jnp.float32)
    res = gmax + mo + (jnp.log(u) - jnp.log(jnp.float32(_K))) / _EPS
    o_ref[0] = res.reshape(_I, _HB, _W)


def kernel(x, offsets):
    n, ch, h, w = x.shape
    offt = (offsets.reshape(_I, _C, _KH * _KW)
            .transpose(0, 2, 1).reshape(_I, _K))
    return pl.pallas_call(
        _mex_kernel,
        out_shape=jax.ShapeDtypeStruct((n, _I, h, w), jnp.float32),
        grid=(n, _NCH),
        in_specs=[
            pl.BlockSpec((1, ch, h, w), lambda i, j: (i, 0, 0, 0)),
            pl.BlockSpec((_I, _K), lambda i, j: (0, 0)),
        ],
        out_specs=pl.BlockSpec((1, _I, _HB, _W), lambda i, j: (i, 0, j, 0)),
        scratch_shapes=[pltpu.VMEM((_C, _M + 2 * _G), jnp.float32)],
        compiler_params=pltpu.CompilerParams(
            dimension_semantics=("parallel", "arbitrary"),
            vmem_limit_bytes=56 * 1024 * 1024,
        ),
        name="mex_pool",
    )(x, offt)
